# Initial kernel scaffold; baseline (speedup 1.0000x reference)
import jax
import jax.numpy as jnp
from jax.experimental import pallas as pl


def kernel(xi, W_img, b_img, W_g, b_g, W_e, b_e):
    raise NotImplementedError("write your pallas kernel here")


# speedup vs baseline: 20.9469x; 20.9469x over previous
import functools

import jax
import jax.numpy as jnp
from jax import lax
from jax.experimental import pallas as pl

B, N, F, D, K = 8, 576, 192, 256, 10
_NEG_BIG = 1e30


def _dot(a, b):
    return lax.dot_general(a, b, (((1,), (0,)), ((), ())),
                           preferred_element_type=jnp.float32)


def _dot_t(a, b):
    return lax.dot_general(a, b, (((1,), (1,)), ((), ())),
                           preferred_element_type=jnp.float32)


def _fused_body(xi_ref, wimg_ref, bimg_ref, wg_ref, bg_ref, we_ref, be_ref,
                xie_ref, gep_ref, out_ref):
    x = xi_ref[0]
    xie = _dot(x, wimg_ref[...]) + bimg_ref[...]
    xie_ref[0] = xie

    sq = jnp.sum(xie * xie, axis=1, keepdims=True)
    gram = _dot_t(xie, xie)
    d2 = sq + jnp.reshape(sq, (1, N)) - 2.0 * gram

    row = lax.broadcasted_iota(jnp.int32, (N, N), 0)
    col = lax.broadcasted_iota(jnp.int32, (N, N), 1)
    d2 = jnp.where(row == col, d2 + 1e9, d2)

    colf = col.astype(jnp.float32)
    adj = jnp.zeros((N, N), dtype=jnp.float32)
    cur = d2
    for _ in range(K):
        rowmin = jnp.min(cur, axis=1, keepdims=True)
        cand = jnp.where(cur == rowmin, colf, jnp.float32(N))
        idx = jnp.min(cand, axis=1, keepdims=True)
        onehot = colf == idx
        adj = adj + onehot.astype(jnp.float32)
        cur = jnp.where(onehot, _NEG_BIG, cur)

    agg = _dot(adj, xie)
    h = xie + agg / jnp.float32(K)
    gep = jnp.maximum(_dot(h, wg_ref[...]) + bg_ref[...], 0.0)
    gep_ref[0] = gep

    s = jnp.sum(gep * we_ref[...], axis=0, keepdims=True)
    i = pl.program_id(0)
    out_ref[pl.ds(i, 1), :] = jnp.sum(s, axis=1, keepdims=True) + be_ref[...]


@functools.partial(jax.jit, static_argnames=("interpret",))
def _run(xi, W_img, b_img2, W_g, b_g2, W_e2, b_e2, interpret=False):
    grid = (B,)
    xie, gep, out = pl.pallas_call(
        _fused_body,
        grid=grid,
        in_specs=[
            pl.BlockSpec((1, N, F), lambda b: (b, 0, 0)),
            pl.BlockSpec((F, D), lambda b: (0, 0)),
            pl.BlockSpec((1, D), lambda b: (0, 0)),
            pl.BlockSpec((D, D), lambda b: (0, 0)),
            pl.BlockSpec((1, D), lambda b: (0, 0)),
            pl.BlockSpec((N, D), lambda b: (0, 0)),
            pl.BlockSpec((1, 1), lambda b: (0, 0)),
        ],
        out_specs=[
            pl.BlockSpec((1, N, D), lambda b: (b, 0, 0)),
            pl.BlockSpec((1, N, D), lambda b: (b, 0, 0)),
            pl.BlockSpec((B, 1), lambda b: (0, 0)),
        ],
        out_shape=[
            jax.ShapeDtypeStruct((B, N, D), jnp.float32),
            jax.ShapeDtypeStruct((B, N, D), jnp.float32),
            jax.ShapeDtypeStruct((B, 1), jnp.float32),
        ],
        interpret=interpret,
    )(xi, W_img, b_img2, W_g, b_g2, W_e2, b_e2)
    return xie, gep, out


def kernel(xi, W_img, b_img, W_g, b_g, W_e, b_e):
    b_img2 = jnp.reshape(b_img, (1, D))
    b_g2 = jnp.reshape(b_g, (1, D))
    W_e2 = jnp.reshape(W_e, (N, D))
    b_e2 = jnp.reshape(b_e, (1, 1))
    return _run(xi, W_img, b_img2, W_g, b_g2, W_e2, b_e2)


# speedup vs baseline: 24.1768x; 1.1542x over previous
import functools

import jax
import jax.numpy as jnp
from jax import lax
from jax.experimental import pallas as pl

B, N, F, D, K = 8, 576, 192, 256, 10
_NEG_BIG = 1e30


def _dot(a, b):
    return lax.dot_general(a, b, (((1,), (0,)), ((), ())),
                           preferred_element_type=jnp.float32)


def _dot_t(a, b):
    return lax.dot_general(a, b, (((1,), (1,)), ((), ())),
                           preferred_element_type=jnp.float32)


def _fused_body(xi_ref, wimg_ref, bimg_ref, wg_ref, bg_ref, we_ref, be_ref,
                xie_ref, gep_ref, out_ref):
    x = xi_ref[0]
    xie = _dot(x, wimg_ref[...]) + bimg_ref[...]
    xie_ref[0] = xie

    sq = jnp.sum(xie * xie, axis=1, keepdims=True)
    gram = _dot_t(xie, xie)
    d2 = sq + jnp.reshape(sq, (1, N)) - 2.0 * gram

    row = lax.broadcasted_iota(jnp.int32, (N, N), 0)
    col = lax.broadcasted_iota(jnp.int32, (N, N), 1)
    d2 = jnp.where(row == col, d2 + 1e9, d2)

    colf = col.astype(jnp.float32)
    cur = d2
    for _ in range(K):
        rowmin = jnp.min(cur, axis=1, keepdims=True)
        cand = jnp.where(cur == rowmin, colf, jnp.float32(N))
        idx = jnp.min(cand, axis=1, keepdims=True)
        cur = jnp.where(colf == idx, _NEG_BIG, cur)

    adj = jnp.where(cur >= 1e29, 1.0, 0.0).astype(jnp.float32)

    agg = _dot(adj, xie)
    h = xie + agg / jnp.float32(K)
    gep = jnp.maximum(_dot(h, wg_ref[...]) + bg_ref[...], 0.0)
    gep_ref[0] = gep

    s = jnp.sum(gep * we_ref[...], axis=0, keepdims=True)
    i = pl.program_id(0)
    out_ref[pl.ds(i, 1), :] = jnp.sum(s, axis=1, keepdims=True) + be_ref[...]


@functools.partial(jax.jit, static_argnames=("interpret",))
def _run(xi, W_img, b_img2, W_g, b_g2, W_e2, b_e2, interpret=False):
    grid = (B,)
    xie, gep, out = pl.pallas_call(
        _fused_body,
        grid=grid,
        in_specs=[
            pl.BlockSpec((1, N, F), lambda b: (b, 0, 0)),
            pl.BlockSpec((F, D), lambda b: (0, 0)),
            pl.BlockSpec((1, D), lambda b: (0, 0)),
            pl.BlockSpec((D, D), lambda b: (0, 0)),
            pl.BlockSpec((1, D), lambda b: (0, 0)),
            pl.BlockSpec((N, D), lambda b: (0, 0)),
            pl.BlockSpec((1, 1), lambda b: (0, 0)),
        ],
        out_specs=[
            pl.BlockSpec((1, N, D), lambda b: (b, 0, 0)),
            pl.BlockSpec((1, N, D), lambda b: (b, 0, 0)),
            pl.BlockSpec((B, 1), lambda b: (0, 0)),
        ],
        out_shape=[
            jax.ShapeDtypeStruct((B, N, D), jnp.float32),
            jax.ShapeDtypeStruct((B, N, D), jnp.float32),
            jax.ShapeDtypeStruct((B, 1), jnp.float32),
        ],
        interpret=interpret,
    )(xi, W_img, b_img2, W_g, b_g2, W_e2, b_e2)
    return xie, gep, out


def kernel(xi, W_img, b_img, W_g, b_g, W_e, b_e):
    b_img2 = jnp.reshape(b_img, (1, D))
    b_g2 = jnp.reshape(b_g, (1, D))
    W_e2 = jnp.reshape(W_e, (N, D))
    b_e2 = jnp.reshape(b_e, (1, 1))
    return _run(xi, W_img, b_img2, W_g, b_g2, W_e2, b_e2)
